# baseline (speedup 1.0000x reference)
import functools

import jax
import jax.numpy as jnp
from jax import lax
from jax.experimental import pallas as pl
from jax.experimental.pallas import tpu as pltpu
from jax.experimental.pallas import tpu_sc as plsc

VOCAB = 21
HIDDEN = 256
MAX_POS = 196
BATCH = 1024
SEQ = 196
EPS = 1e-12

NPM = 8
NTAIL = 4
NMAIN = SEQ // NPM
NB = 8
TROWS_PAD = 176


def _rsqrt16(x):
    i = lax.bitcast_convert_type(x, jnp.int32)
    y = lax.bitcast_convert_type(jnp.int32(0x5F3759DF) - (i >> 1), jnp.float32)
    for _ in range(4):
        y = y * (1.5 - 0.5 * x * y * y)
    return y


def _splat(v):
    return jnp.full((16,), v, jnp.int32)


def kernel(input_ids, word_emb, pos_emb, gamma, beta):
    ids_t = input_ids.astype(jnp.int32).T
    gam2 = gamma.reshape(2, 128)
    bet2 = beta.reshape(2, 128)

    mesh = plsc.VectorSubcoreMesh(core_axis_name="c", subcore_axis_name="s")

    @functools.partial(
        pl.kernel,
        out_type=jax.ShapeDtypeStruct((BATCH, SEQ, HIDDEN), jnp.float32),
        mesh=mesh,
        compiler_params=pltpu.CompilerParams(needs_layout_passes=False),
        scratch_types=[
            pltpu.VMEM((VOCAB, HIDDEN), jnp.float32),
            pltpu.VMEM((NPM, HIDDEN), jnp.float32),
            pltpu.VMEM((NTAIL, HIDDEN), jnp.float32),
            pltpu.VMEM((2, 128), jnp.float32),
            pltpu.VMEM((2, 128), jnp.float32),
            pltpu.VMEM((NPM, BATCH), jnp.int32),
            pltpu.VMEM((NTAIL, BATCH), jnp.int32),
            pltpu.VMEM((TROWS_PAD, HIDDEN), jnp.float32),
            pltpu.VMEM((NB, NPM, HIDDEN), jnp.float32),
            pltpu.VMEM((NB, NPM, HIDDEN), jnp.float32),
            pltpu.VMEM((NB, NTAIL, HIDDEN), jnp.float32),
            pltpu.VMEM((NB, NTAIL, HIDDEN), jnp.float32),
            pltpu.SemaphoreType.DMA,
            pltpu.SemaphoreType.DMA,
        ],
    )
    def run(ids_hbm, word_hbm, pos_hbm, gam_hbm, bet_hbm, out_hbm,
            word_v, pos_v, pos_tv, gam_v, bet_v, ids_v, ids_tv, tab_v,
            buf0, buf1, buf0t, buf1t, sem0, sem1):
        wid = lax.axis_index("s") * 2 + lax.axis_index("c")
        lanes = lax.iota(jnp.int32, 16)

        def worker(np_, p0, pos_vr, ids_vr, bufA, bufB):
            p0 = pl.multiple_of(p0, 8)
            pltpu.sync_copy(word_hbm, word_v)
            pltpu.sync_copy(pos_hbm.at[pl.ds(p0, np_)], pos_vr)
            pltpu.sync_copy(gam_hbm, gam_v)
            pltpu.sync_copy(bet_hbm, bet_v)
            pltpu.sync_copy(ids_hbm.at[pl.ds(p0, np_)], ids_vr)

            nrows = np_ * VOCAB
            ngroups = (nrows + 15) // 16

            def build_group(g, _):
                r = g * 16 + lanes
                rc = jnp.minimum(r, nrows - 1)
                pidx = rc // VOCAB
                widx = rc % VOCAB

                zero = jnp.zeros((16,), jnp.float32)

                @plsc.parallel_loop(0, HIDDEN, unroll=8, carry=(zero, zero))
                def pass1(h, carry):
                    s, ss = carry
                    hv = (_splat(h) + lanes) & (HIDDEN - 1)
                    w = plsc.load_gather(word_v, [widx, hv])
                    p = plsc.load_gather(pos_vr, [pidx, hv])
                    x = w + p
                    plsc.store_scatter(tab_v, [r, hv], x)
                    return s + x, ss + x * x

                s, ss = pass1
                mean = s * (1.0 / HIDDEN)
                var = ss * (1.0 / HIDDEN) - mean * mean
                inv = _rsqrt16(var + EPS)

                @plsc.parallel_loop(0, HIDDEN, unroll=8)
                def pass2(h):
                    hv = (_splat(h) + lanes) & (HIDDEN - 1)
                    x = plsc.load_gather(tab_v, [r, hv])
                    gh = plsc.load_gather(gam_v, [hv >> 7, hv & 127])
                    bh = plsc.load_gather(bet_v, [hv >> 7, hv & 127])
                    plsc.store_scatter(tab_v, [r, hv], (x - mean) * inv * gh + bh)

                return 0

            lax.fori_loop(0, ngroups, build_group, 0)

            def fill_block(j, buf):
                b0 = j * NB

                @plsc.parallel_loop(0, (NB * np_) // 16)
                def group(g):
                    t = g * 16 + lanes
                    bl = t // np_
                    pi = t % np_
                    ids16 = plsc.load_gather(ids_vr, [pi, b0 + bl])
                    row = pi * VOCAB + ids16

                    @plsc.parallel_loop(0, HIDDEN, unroll=32)
                    def hloop(h):
                        hv = (_splat(h) + lanes) & (HIDDEN - 1)
                        v = plsc.load_gather(tab_v, [row, hv])
                        plsc.store_scatter(buf, [bl, pi, hv], v)

            def out_slice(j):
                return out_hbm.at[pl.ds(j * NB, NB), pl.ds(p0, np_), :]

            def block_pair(jj, _):
                j0 = 2 * jj
                j1 = 2 * jj + 1

                @pl.when(jj > 0)
                def _():
                    pltpu.make_async_copy(bufA, out_slice(j0), sem0).wait()

                fill_block(j0, bufA)
                pltpu.async_copy(bufA, out_slice(j0), sem0)

                @pl.when(jj > 0)
                def _():
                    pltpu.make_async_copy(bufB, out_slice(j1), sem1).wait()

                fill_block(j1, bufB)
                pltpu.async_copy(bufB, out_slice(j1), sem1)
                return 0

            nblk = BATCH // NB
            lax.fori_loop(0, nblk // 2, block_pair, 0)
            pltpu.make_async_copy(bufA, out_slice(nblk - 2), sem0).wait()
            pltpu.make_async_copy(bufB, out_slice(nblk - 1), sem1).wait()

        @pl.when(wid < NMAIN)
        def _():
            worker(NPM, wid * NPM, pos_v, ids_v, buf0, buf1)

        @pl.when(wid == NMAIN)
        def _():
            worker(NTAIL, NMAIN * NPM, pos_tv, ids_tv, buf0t, buf1t)

    return run(ids_t, word_emb, pos_emb, gam2, bet2)


# speedup vs baseline: 1.0207x; 1.0207x over previous
import functools

import jax
import jax.numpy as jnp
from jax import lax
from jax.experimental import pallas as pl
from jax.experimental.pallas import tpu as pltpu
from jax.experimental.pallas import tpu_sc as plsc

VOCAB = 21
HIDDEN = 256
MAX_POS = 196
BATCH = 1024
SEQ = 196
EPS = 1e-12

NPM = 8
NTAIL = 4
NMAIN = SEQ // NPM
NB = 4
NBUF = 4
TROWS_PAD = 176


def _rsqrt16(x):
    i = lax.bitcast_convert_type(x, jnp.int32)
    y = lax.bitcast_convert_type(jnp.int32(0x5F3759DF) - (i >> 1), jnp.float32)
    for _ in range(4):
        y = y * (1.5 - 0.5 * x * y * y)
    return y


def _splat(v):
    return jnp.full((16,), v, jnp.int32)


def kernel(input_ids, word_emb, pos_emb, gamma, beta):
    ids_t = input_ids.astype(jnp.int32).T
    gam2 = gamma.reshape(2, 128)
    bet2 = beta.reshape(2, 128)

    mesh = plsc.VectorSubcoreMesh(core_axis_name="c", subcore_axis_name="s")

    @functools.partial(
        pl.kernel,
        out_type=jax.ShapeDtypeStruct((BATCH, SEQ, HIDDEN), jnp.float32),
        mesh=mesh,
        compiler_params=pltpu.CompilerParams(needs_layout_passes=False),
        scratch_types=[
            pltpu.VMEM((VOCAB, HIDDEN), jnp.float32),
            pltpu.VMEM((NPM, HIDDEN), jnp.float32),
            pltpu.VMEM((NTAIL, HIDDEN), jnp.float32),
            pltpu.VMEM((2, 128), jnp.float32),
            pltpu.VMEM((2, 128), jnp.float32),
            pltpu.VMEM((NPM, BATCH), jnp.int32),
            pltpu.VMEM((NTAIL, BATCH), jnp.int32),
            pltpu.VMEM((TROWS_PAD, HIDDEN), jnp.float32),
            [pltpu.VMEM((NB, NPM, HIDDEN), jnp.float32) for _ in range(NBUF)],
            [pltpu.VMEM((NB, NTAIL, HIDDEN), jnp.float32) for _ in range(NBUF)],
            [pltpu.SemaphoreType.DMA for _ in range(NBUF)],
        ],
    )
    def run(ids_hbm, word_hbm, pos_hbm, gam_hbm, bet_hbm, out_hbm,
            word_v, pos_v, pos_tv, gam_v, bet_v, ids_v, ids_tv, tab_v,
            bufs, bufs_t, sems):
        wid = lax.axis_index("s") * 2 + lax.axis_index("c")
        lanes = lax.iota(jnp.int32, 16)

        def worker(np_, p0, pos_vr, ids_vr, bufs_):
            p0 = pl.multiple_of(p0, 8)
            pltpu.sync_copy(word_hbm, word_v)
            pltpu.sync_copy(pos_hbm.at[pl.ds(p0, np_)], pos_vr)
            pltpu.sync_copy(gam_hbm, gam_v)
            pltpu.sync_copy(bet_hbm, bet_v)
            pltpu.sync_copy(ids_hbm.at[pl.ds(p0, np_)], ids_vr)

            nrows = np_ * VOCAB
            ngroups = (nrows + 15) // 16

            def build_group(g, _):
                r = g * 16 + lanes
                rc = jnp.minimum(r, nrows - 1)
                pidx = rc // VOCAB
                widx = rc % VOCAB

                zero = jnp.zeros((16,), jnp.float32)

                @plsc.parallel_loop(0, HIDDEN, unroll=8, carry=(zero, zero))
                def pass1(h, carry):
                    s, ss = carry
                    hv = (_splat(h) + lanes) & (HIDDEN - 1)
                    w = plsc.load_gather(word_v, [widx, hv])
                    p = plsc.load_gather(pos_vr, [pidx, hv])
                    x = w + p
                    plsc.store_scatter(tab_v, [r, hv], x)
                    return s + x, ss + x * x

                s, ss = pass1
                mean = s * (1.0 / HIDDEN)
                var = ss * (1.0 / HIDDEN) - mean * mean
                inv = _rsqrt16(var + EPS)

                @plsc.parallel_loop(0, HIDDEN, unroll=8)
                def pass2(h):
                    hv = (_splat(h) + lanes) & (HIDDEN - 1)
                    x = plsc.load_gather(tab_v, [r, hv])
                    gh = plsc.load_gather(gam_v, [hv >> 7, hv & 127])
                    bh = plsc.load_gather(bet_v, [hv >> 7, hv & 127])
                    plsc.store_scatter(tab_v, [r, hv], (x - mean) * inv * gh + bh)

                return 0

            lax.fori_loop(0, ngroups, build_group, 0)

            def fill_block(j, buf):
                b0 = j * NB

                def group(g, _):
                    t = g * 16 + lanes
                    bl = t // np_
                    pi = t % np_
                    ids16 = plsc.load_gather(ids_vr, [pi, b0 + bl])
                    row = pi * VOCAB + ids16

                    @plsc.parallel_loop(0, HIDDEN, unroll=16)
                    def hloop(h):
                        hv = (_splat(h) + lanes) & (HIDDEN - 1)
                        v = plsc.load_gather(tab_v, [row, hv])
                        plsc.store_scatter(buf, [bl, pi, hv], v)

                    return 0

                lax.fori_loop(0, (NB * np_) // 16, group, 0)

            def out_slice(j):
                return out_hbm.at[pl.ds(j * NB, NB), pl.ds(p0, np_), :]

            nblk = BATCH // NB

            def block_ring(jj, _):
                for k in range(NBUF):
                    j = NBUF * jj + k

                    @pl.when(jj > 0)
                    def _():
                        pltpu.make_async_copy(
                            bufs_[k], out_slice(j - NBUF), sems[k]).wait()

                    fill_block(j, bufs_[k])
                    pltpu.async_copy(bufs_[k], out_slice(j), sems[k])
                return 0

            lax.fori_loop(0, nblk // NBUF, block_ring, 0)
            for k in range(NBUF):
                pltpu.make_async_copy(
                    bufs_[k], out_slice(nblk - NBUF + k), sems[k]).wait()

        @pl.when(wid < NMAIN)
        def _():
            worker(NPM, wid * NPM, pos_v, ids_v, bufs)

        @pl.when(wid == NMAIN)
        def _():
            worker(NTAIL, NMAIN * NPM, pos_tv, ids_tv, bufs_t)

    return run(ids_t, word_emb, pos_emb, gam2, bet2)


# speedup vs baseline: 1.3246x; 1.2977x over previous
import functools

import jax
import jax.numpy as jnp
from jax import lax
from jax.experimental import pallas as pl
from jax.experimental.pallas import tpu as pltpu
from jax.experimental.pallas import tpu_sc as plsc

VOCAB = 21
HIDDEN = 256
MAX_POS = 196
BATCH = 1024
SEQ = 196
EPS = 1e-12

NPM = 8
NTAIL = 4
NMAIN = SEQ // NPM
NB = 8
TROWS_PAD = 176


def _rsqrt16(x):
    i = lax.bitcast_convert_type(x, jnp.int32)
    y = lax.bitcast_convert_type(jnp.int32(0x5F3759DF) - (i >> 1), jnp.float32)
    for _ in range(4):
        y = y * (1.5 - 0.5 * x * y * y)
    return y


def _splat(v):
    return jnp.full((16,), v, jnp.int32)


def kernel(input_ids, word_emb, pos_emb, gamma, beta):
    ids_t = input_ids.astype(jnp.int32).T
    gam2 = gamma.reshape(2, 128)
    bet2 = beta.reshape(2, 128)

    mesh = plsc.VectorSubcoreMesh(core_axis_name="c", subcore_axis_name="s")

    @functools.partial(
        pl.kernel,
        out_type=jax.ShapeDtypeStruct((BATCH, SEQ, HIDDEN), jnp.float32),
        mesh=mesh,
        compiler_params=pltpu.CompilerParams(needs_layout_passes=False),
        scratch_types=[
            pltpu.VMEM((VOCAB, HIDDEN), jnp.float32),
            pltpu.VMEM((NPM, HIDDEN), jnp.float32),
            pltpu.VMEM((NTAIL, HIDDEN), jnp.float32),
            pltpu.VMEM((2, 128), jnp.float32),
            pltpu.VMEM((2, 128), jnp.float32),
            pltpu.VMEM((NPM, BATCH), jnp.int32),
            pltpu.VMEM((NTAIL, BATCH), jnp.int32),
            pltpu.VMEM((TROWS_PAD, HIDDEN), jnp.float32),
            pltpu.VMEM((NB, NPM, HIDDEN), jnp.float32),
            pltpu.VMEM((NB, NPM, HIDDEN), jnp.float32),
            pltpu.VMEM((NB, NTAIL, HIDDEN), jnp.float32),
            pltpu.VMEM((NB, NTAIL, HIDDEN), jnp.float32),
            pltpu.SemaphoreType.DMA,
            pltpu.SemaphoreType.DMA,
        ],
    )
    def run(ids_hbm, word_hbm, pos_hbm, gam_hbm, bet_hbm, out_hbm,
            word_v, pos_v, pos_tv, gam_v, bet_v, ids_v, ids_tv, tab_v,
            buf0, buf1, buf0t, buf1t, sem0, sem1):
        wid = lax.axis_index("s") * 2 + lax.axis_index("c")
        lanes = lax.iota(jnp.int32, 16)

        def worker(np_, p0, pos_vr, ids_vr, bufA, bufB):
            p0 = pl.multiple_of(p0, 8)
            pltpu.sync_copy(word_hbm, word_v)
            pltpu.sync_copy(pos_hbm.at[pl.ds(p0, np_)], pos_vr)
            pltpu.sync_copy(gam_hbm, gam_v)
            pltpu.sync_copy(bet_hbm, bet_v)
            pltpu.sync_copy(ids_hbm.at[pl.ds(p0, np_)], ids_vr)

            nrows = np_ * VOCAB
            ngroups = (nrows + 15) // 16

            def build_group(g, _):
                r = g * 16 + lanes
                rc = jnp.minimum(r, nrows - 1)
                pidx = rc // VOCAB
                widx = rc % VOCAB

                zero = jnp.zeros((16,), jnp.float32)

                @plsc.parallel_loop(0, HIDDEN, unroll=8, carry=(zero, zero))
                def pass1(h, carry):
                    s, ss = carry
                    hv = (_splat(h) + lanes) & (HIDDEN - 1)
                    w = plsc.load_gather(word_v, [widx, hv])
                    p = plsc.load_gather(pos_vr, [pidx, hv])
                    x = w + p
                    plsc.store_scatter(tab_v, [r, hv], x)
                    return s + x, ss + x * x

                s, ss = pass1
                mean = s * (1.0 / HIDDEN)
                var = ss * (1.0 / HIDDEN) - mean * mean
                inv = _rsqrt16(var + EPS)

                @plsc.parallel_loop(0, HIDDEN, unroll=8)
                def pass2(h):
                    hv = (_splat(h) + lanes) & (HIDDEN - 1)
                    x = plsc.load_gather(tab_v, [r, hv])
                    gh = plsc.load_gather(gam_v, [hv >> 7, hv & 127])
                    bh = plsc.load_gather(bet_v, [hv >> 7, hv & 127])
                    plsc.store_scatter(tab_v, [r, hv], (x - mean) * inv * gh + bh)

                return 0

            lax.fori_loop(0, ngroups, build_group, 0)

            ngr = (NB * np_) // 16

            def fill_block(j, buf):
                b0 = j * NB
                rows = []
                bls = []
                pis = []
                for g in range(ngr):
                    t = g * 16 + lanes
                    bl = t // np_
                    pi = t % np_
                    ids16 = plsc.load_gather(ids_vr, [pi, b0 + bl])
                    rows.append(pi * VOCAB + ids16)
                    bls.append(bl)
                    pis.append(pi)

                @plsc.parallel_loop(0, HIDDEN, unroll=4)
                def hloop(h):
                    hv = (_splat(h) + lanes) & (HIDDEN - 1)
                    for g in range(ngr):
                        v = plsc.load_gather(tab_v, [rows[g], hv])
                        plsc.store_scatter(buf, [bls[g], pis[g], hv], v)

            def out_slice(j):
                return out_hbm.at[pl.ds(j * NB, NB), pl.ds(p0, np_), :]

            def block_pair(jj, _):
                j0 = 2 * jj
                j1 = 2 * jj + 1

                @pl.when(jj > 0)
                def _():
                    pltpu.make_async_copy(bufA, out_slice(j0), sem0).wait()

                fill_block(j0, bufA)
                pltpu.async_copy(bufA, out_slice(j0), sem0)

                @pl.when(jj > 0)
                def _():
                    pltpu.make_async_copy(bufB, out_slice(j1), sem1).wait()

                fill_block(j1, bufB)
                pltpu.async_copy(bufB, out_slice(j1), sem1)
                return 0

            nblk = BATCH // NB
            lax.fori_loop(0, nblk // 2, block_pair, 0)
            pltpu.make_async_copy(bufA, out_slice(nblk - 2), sem0).wait()
            pltpu.make_async_copy(bufB, out_slice(nblk - 1), sem1).wait()

        @pl.when(wid < NMAIN)
        def _():
            worker(NPM, wid * NPM, pos_v, ids_v, buf0, buf1)

        @pl.when(wid == NMAIN)
        def _():
            worker(NTAIL, NMAIN * NPM, pos_tv, ids_tv, buf0t, buf1t)

    return run(ids_t, word_emb, pos_emb, gam2, bet2)
